# baseline (speedup 1.0000x reference)
import functools

import jax
import jax.numpy as jnp
from jax import lax
from jax.experimental import pallas as pl
from jax.experimental.pallas import tpu as pltpu
from jax.experimental.pallas import tpu_sc as plsc

H = 1024
N = 16384
NC, NS, L = 2, 16, 16
NW = NC * NS
PER_W = N // NW
C = 32
NCHUNK = PER_W // C
VREGS_PER_ROW = H // L


def _body(hid_hbm, idx_hbm, tab_hbm, out_hbm, idx_v, pos_v, hid_v, gsem, hsem):
    wid = lax.axis_index("s") * NC + lax.axis_index("c")
    base = wid * PER_W
    pltpu.sync_copy(idx_hbm.at[pl.ds(base, PER_W)], idx_v)

    def chunk(c, _):
        row0 = base + c * C
        gcp = pltpu.async_copy(tab_hbm.at[idx_v.at[pl.ds(c * C, C)]], pos_v, gsem)
        hcp = pltpu.async_copy(hid_hbm.at[pl.ds(row0, C)], hid_v, hsem)
        gcp.wait()
        hcp.wait()

        def row(j, _):
            def vec(i, _):
                sl = pl.ds(i * L, L)
                hid_v[j, sl] = hid_v[j, sl] + pos_v[j, sl]
                return 0

            return lax.fori_loop(0, VREGS_PER_ROW, vec, 0)

        lax.fori_loop(0, C, row, 0)
        pltpu.sync_copy(hid_v, out_hbm.at[pl.ds(row0, C)])
        return 0

    lax.fori_loop(0, NCHUNK, chunk, 0)


@functools.partial(jax.jit, static_argnames=())
def _run(hidden2d, idx, tab):
    mesh = plsc.VectorSubcoreMesh(core_axis_name="c", subcore_axis_name="s")
    k = pl.kernel(
        _body,
        out_type=jax.ShapeDtypeStruct((N, H), jnp.float32),
        mesh=mesh,
        scratch_types=[
            pltpu.VMEM((PER_W,), jnp.int32),
            pltpu.VMEM((C, H), jnp.float32),
            pltpu.VMEM((C, H), jnp.float32),
            pltpu.SemaphoreType.DMA,
            pltpu.SemaphoreType.DMA,
        ],
    )
    return k(hidden2d, idx, tab)


def kernel(hidden, position_ids, pos_embed):
    B, T, Hh = hidden.shape
    hidden2d = hidden.reshape(B * T, Hh)
    idx = position_ids.reshape(B * T).astype(jnp.int32)
    out = _run(hidden2d, idx, pos_embed)
    return out.reshape(B, T, Hh)


# speedup vs baseline: 1.1446x; 1.1446x over previous
import functools

import jax
import jax.numpy as jnp
from jax import lax
from jax.experimental import pallas as pl
from jax.experimental.pallas import tpu as pltpu
from jax.experimental.pallas import tpu_sc as plsc

H = 1024
N = 16384
NC, NS, L = 2, 16, 16
NW = NC * NS
PER_W = N // NW
C = 16
NCHUNK = PER_W // C
VREGS_PER_ROW = H // L


def _body(hid_hbm, idx_hbm, tab_hbm, out_hbm, idx_v, pos_v, hid_v,
          gs0, gs1, hs0, hs1, os0, os1):
    wid = lax.axis_index("s") * NC + lax.axis_index("c")
    base = wid * PER_W
    pltpu.sync_copy(idx_hbm.at[pl.ds(base, PER_W)], idx_v)
    gsem = (gs0, gs1)
    hsem = (hs0, hs1)
    osem = (os0, os1)

    def issue(c):
        s = c % 2
        g = pltpu.async_copy(tab_hbm.at[idx_v.at[pl.ds(c * C, C)]],
                             pos_v.at[s], gsem[s])
        h = pltpu.async_copy(hid_hbm.at[pl.ds(base + c * C, C)],
                             hid_v.at[s], hsem[s])
        return g, h

    def add_chunk(s):
        def row(j, _):
            def vec(i, _):
                sl = pl.ds(i * L, L)
                hid_v[s, j, sl] = hid_v[s, j, sl] + pos_v[s, j, sl]
                return 0

            return lax.fori_loop(0, VREGS_PER_ROW, vec, 0, unroll=16)

        lax.fori_loop(0, C, row, 0)

    pend = [None, None]
    out_desc = [None, None]
    pend[0] = issue(0)
    for c in range(NCHUNK):
        s = c % 2
        o = 1 - s
        if c + 1 < NCHUNK:
            if out_desc[o] is not None:
                out_desc[o].wait()
            pend[o] = issue(c + 1)
        gd, hd = pend[s]
        gd.wait()
        hd.wait()
        add_chunk(s)
        out_desc[s] = pltpu.async_copy(hid_v.at[s],
                                       out_hbm.at[pl.ds(base + c * C, C)],
                                       osem[s])
    out_desc[0].wait()
    out_desc[1].wait()


@jax.jit
def _run(hidden2d, idx, tab):
    mesh = plsc.VectorSubcoreMesh(core_axis_name="c", subcore_axis_name="s")
    k = pl.kernel(
        _body,
        out_type=jax.ShapeDtypeStruct((N, H), jnp.float32),
        mesh=mesh,
        scratch_types=[
            pltpu.VMEM((PER_W,), jnp.int32),
            pltpu.VMEM((2, C, H), jnp.float32),
            pltpu.VMEM((2, C, H), jnp.float32),
            pltpu.SemaphoreType.DMA,
            pltpu.SemaphoreType.DMA,
            pltpu.SemaphoreType.DMA,
            pltpu.SemaphoreType.DMA,
            pltpu.SemaphoreType.DMA,
            pltpu.SemaphoreType.DMA,
        ],
    )
    return k(hidden2d, idx, tab)


def kernel(hidden, position_ids, pos_embed):
    B, T, Hh = hidden.shape
    hidden2d = hidden.reshape(B * T, Hh)
    idx = position_ids.reshape(B * T).astype(jnp.int32)
    out = _run(hidden2d, idx, pos_embed)
    return out.reshape(B, T, Hh)
